# Initial kernel scaffold; baseline (speedup 1.0000x reference)
import jax
import jax.numpy as jnp
from jax.experimental import pallas as pl


def kernel(node_feat, edge_feat, global_feat, params, edge_idx, node_size):
    raise NotImplementedError("write your pallas kernel here")


# speedup vs baseline: 1.9806x; 1.9806x over previous
import functools

import jax
import jax.numpy as jnp
from jax import lax
from jax.experimental import pallas as pl
from jax.experimental.pallas import tpu as pltpu
from jax.experimental.pallas import tpu_sc as plsc

F32 = jnp.float32
_NT = 16
_SUP = 1024
_IDXG = 128


def _ln(x, g, b):
    m = jnp.mean(x, axis=-1, keepdims=True)
    v = jnp.mean(jnp.square(x - m), axis=-1, keepdims=True)
    return (x - m) * lax.rsqrt(v + 1e-5) * g + b


def _relu(x):
    return jnp.maximum(x, 0.0)


def _dot(x, w):
    return jnp.dot(x.astype(jnp.bfloat16), w.astype(jnp.bfloat16),
                   preferred_element_type=F32)


def _enc_body(x_ref, w0, b0, w1, b1, w2, b2, g, b, o_ref):
    h = _relu(_dot(x_ref[...], w0[...]) + b0[...])
    h = _relu(_dot(h, w1[...]) + b1[...])
    h = _dot(h, w2[...]) + b2[...]
    o_ref[...] = _ln(h, g[...], b[...])


def _full(shape):
    return pl.BlockSpec(shape, lambda i: tuple(0 for _ in shape))


def _encoder(x, p, rb):
    n, din = x.shape
    w0, w1, w2 = p["W"]
    b0, b1, b2 = (v.reshape(1, -1) for v in p["b"])
    g = p["ln_g"].reshape(1, -1)
    b = p["ln_b"].reshape(1, -1)
    dout = w2.shape[1]
    return pl.pallas_call(
        _enc_body,
        grid=(n // rb,),
        in_specs=[
            pl.BlockSpec((rb, din), lambda i: (i, 0)),
            _full(w0.shape), _full(b0.shape), _full(w1.shape), _full(b1.shape),
            _full(w2.shape), _full(b2.shape), _full(g.shape), _full(b.shape),
        ],
        out_specs=pl.BlockSpec((rb, dout), lambda i: (i, 0)),
        out_shape=jax.ShapeDtypeStruct((n, dout), F32),
    )(x, w0, b0, w1, b1, w2, b2, g, b)


def _edge_body(e_ref, r_ref, s_ref, w0, b0, w1, b1, w2, b2, g, b,
               proc_ref, newe_ref):
    e = e_ref[...]
    x = jnp.concatenate([e, r_ref[...], s_ref[...]], axis=1)
    h = _relu(_dot(x, w0[...]) + b0[...])
    h = _relu(_dot(h, w1[...]) + b1[...])
    h = _dot(h, w2[...]) + b2[...]
    pz = _ln(h, g[...], b[...])
    proc_ref[...] = pz
    newe_ref[...] = e + pz


def _edge_stage(edge, r_rows, s_rows, p, rb):
    n, d = edge.shape
    w0, w1, w2 = p["W"]
    b0, b1, b2 = (v.reshape(1, -1) for v in p["b"])
    g = p["ln_g"].reshape(1, -1)
    b = p["ln_b"].reshape(1, -1)
    out = jax.ShapeDtypeStruct((n, d), F32)
    return pl.pallas_call(
        _edge_body,
        grid=(n // rb,),
        in_specs=[
            pl.BlockSpec((rb, d), lambda i: (i, 0)),
            pl.BlockSpec((rb, d), lambda i: (i, 0)),
            pl.BlockSpec((rb, d), lambda i: (i, 0)),
            _full(w0.shape), _full(b0.shape), _full(w1.shape), _full(b1.shape),
            _full(w2.shape), _full(b2.shape), _full(g.shape), _full(b.shape),
        ],
        out_specs=[
            pl.BlockSpec((rb, d), lambda i: (i, 0)),
            pl.BlockSpec((rb, d), lambda i: (i, 0)),
        ],
        out_shape=[out, out],
    )(edge, r_rows, s_rows, w0, b0, w1, b1, w2, b2, g, b)


def _node_body(n_ref, a_ref, gl_ref, w0, b0, w1, b1, w2, b2, g, b, newn_ref):
    n = n_ref[...]
    gl = gl_ref[...]
    d = n.shape[1]
    x = (_dot(n, w0[0:d, :]) + _dot(a_ref[...], w0[d:2 * d, :])
         + gl[:, 0:1] * w0[2 * d:2 * d + 1, :]
         + gl[:, 1:2] * w0[2 * d + 1:2 * d + 2, :] + b0[...])
    h = _relu(x)
    h = _relu(_dot(h, w1[...]) + b1[...])
    h = _dot(h, w2[...]) + b2[...]
    newn_ref[...] = n + _ln(h, g[...], b[...])


def _node_stage(node, agg, glob, p, rb):
    n, d = node.shape
    w0, w1, w2 = p["W"]
    b0, b1, b2 = (v.reshape(1, -1) for v in p["b"])
    g = p["ln_g"].reshape(1, -1)
    b = p["ln_b"].reshape(1, -1)
    return pl.pallas_call(
        _node_body,
        grid=(n // rb,),
        in_specs=[
            pl.BlockSpec((rb, d), lambda i: (i, 0)),
            pl.BlockSpec((rb, d), lambda i: (i, 0)),
            pl.BlockSpec((rb, glob.shape[1]), lambda i: (i, 0)),
            _full(w0.shape), _full(b0.shape), _full(w1.shape), _full(b1.shape),
            _full(w2.shape), _full(b2.shape), _full(g.shape), _full(b.shape),
        ],
        out_specs=pl.BlockSpec((rb, d), lambda i: (i, 0)),
        out_shape=jax.ShapeDtypeStruct((n, d), F32),
    )(node, agg, glob, w0, b0, w1, b1, w2, b2, g, b)


def _dec_body(x_ref, w0, b0, w1, b1, w2, b2, o_ref):
    h = _relu(_dot(x_ref[...], w0[...]) + b0[...])
    h = _relu(_dot(h, w1[...]) + b1[...])
    o_ref[...] = _dot(h, w2[...]) + b2[...]


def _decoder(x, p, rb):
    n, _ = x.shape
    w0, w1, w2 = p["W"]
    b0, b1, b2 = (v.reshape(1, -1) for v in p["b"])
    dout = w2.shape[1]
    return pl.pallas_call(
        _dec_body,
        grid=(n // rb,),
        in_specs=[
            pl.BlockSpec((rb, x.shape[1]), lambda i: (i, 0)),
            _full(w0.shape), _full(b0.shape), _full(w1.shape), _full(b1.shape),
            _full(w2.shape), _full(b2.shape),
        ],
        out_specs=pl.BlockSpec((rb, dout), lambda i: (i, 0)),
        out_shape=jax.ShapeDtypeStruct((n, dout), F32),
    )(x, w0, b0, w1, b1, w2, b2)


@functools.lru_cache(maxsize=None)
def _gather_kernel(n_nodes, e, d):
    per = e // _NT
    nf, tail = divmod(per, _SUP)
    ntg, tg_rem = divmod(tail, _IDXG)
    mesh = plsc.VectorSubcoreMesh(core_axis_name="c", subcore_axis_name="s")
    out = jax.ShapeDtypeStruct((e, d), F32)

    @functools.partial(
        pl.kernel,
        out_type=(out, out),
        mesh=mesh,
        compiler_params=pltpu.CompilerParams(use_tc_tiling_on_sc=False),
        scratch_types=[
            pltpu.VMEM((_SUP,), jnp.int32),
            pltpu.VMEM((_SUP, d), F32),
            pltpu.SemaphoreType.DMA,
        ],
    )
    def k(tab, ridx, sidx, rout, sout, idx_v, rows_v, sem):
        c = lax.axis_index("c")
        s = lax.axis_index("s")
        base = s * per

        def fire(ngroups, rem):
            cps = [
                pltpu.async_copy(
                    tab.at[idx_v.at[pl.ds(gi * _IDXG, _IDXG)]],
                    rows_v.at[pl.ds(gi * _IDXG, _IDXG)], sem)
                for gi in range(ngroups)
            ]
            if rem:
                cps.append(pltpu.async_copy(
                    tab.at[idx_v.at[pl.ds(ngroups * _IDXG, rem)]],
                    rows_v.at[pl.ds(ngroups * _IDXG, rem)], sem))
            for cp in cps:
                cp.wait()

        def run(ih, oh):
            def body(i, _):
                off = base + i * _SUP
                pltpu.sync_copy(ih.at[pl.ds(off, _SUP)], idx_v)
                fire(_SUP // _IDXG, 0)
                pltpu.sync_copy(rows_v, oh.at[pl.ds(off, _SUP)])
                return 0

            lax.fori_loop(0, nf, body, 0)
            if tail:
                off = base + nf * _SUP
                pltpu.sync_copy(ih.at[pl.ds(off, tail)], idx_v.at[pl.ds(0, tail)])
                fire(ntg, tg_rem)
                pltpu.sync_copy(rows_v.at[pl.ds(0, tail)], oh.at[pl.ds(off, tail)])

        pl.when(c == 0)(lambda: run(ridx, rout))
        pl.when(c == 1)(lambda: run(sidx, sout))

    return k


@functools.lru_cache(maxsize=None)
def _scatter_kernel(n_nodes, e, d):
    sup = 384
    per = e // _NT
    nf, tail = divmod(per, sup)
    half = n_nodes // 2
    pad = ((half + 8 + 15) // 16) * 16
    zrows = pad // _NT
    q, r = divmod(half, _NT)
    tail_groups = -(-tail // _IDXG) if tail else 0
    mesh = plsc.VectorSubcoreMesh(core_axis_name="c", subcore_axis_name="s")

    @functools.partial(
        pl.kernel,
        out_type=jax.ShapeDtypeStruct((n_nodes, d), F32),
        mesh=mesh,
        compiler_params=pltpu.CompilerParams(use_tc_tiling_on_sc=False),
        scratch_types=[
            pltpu.VMEM((sup,), jnp.int32),
            pltpu.VMEM((sup // _IDXG, _IDXG), jnp.int32),
            pltpu.VMEM((sup, d), F32),
            pltpu.VMEM_SHARED((pad, d), F32),
            pltpu.SemaphoreType.DMA,
        ],
    )
    def k(idx_hbm, rows_hbm, zero_hbm, out_hbm, raw_v, idx2_v, rows_v, acc, sem):
        c = lax.axis_index("c")
        s = lax.axis_index("s")
        nbase = c * half
        pltpu.sync_copy(zero_hbm.at[pl.ds(s * zrows, zrows)],
                        acc.at[pl.ds(s * zrows, zrows)])
        plsc.subcore_barrier()

        trash = jnp.full((16,), half, jnp.int32)
        ebase = s * per

        def adjust(count):
            def a_body(j, _):
                v = raw_v[pl.ds(j * 16, 16)]
                ok = (v >= nbase) & (v < nbase + half)
                lv = jnp.where(ok, v - nbase, half)
                idx2_v[j // 8, pl.ds((j % 8) * 16, 16)] = lv
                return 0

            lax.fori_loop(0, count // 16, a_body, 0)

        def scatter_groups(ngroups):
            for gi in range(ngroups):
                pltpu.sync_copy(rows_v.at[pl.ds(gi * _IDXG, _IDXG)],
                                acc.at[idx2_v.at[gi]], add=True)

        def body(i, _):
            off = ebase + i * sup
            pltpu.sync_copy(idx_hbm.at[pl.ds(off, sup)], raw_v)
            pltpu.sync_copy(rows_hbm.at[pl.ds(off, sup)], rows_v)
            adjust(sup)
            scatter_groups(sup // _IDXG)
            return 0

        lax.fori_loop(0, nf, body, 0)
        if tail:
            off = ebase + nf * sup
            pltpu.sync_copy(idx_hbm.at[pl.ds(off, tail)], raw_v.at[pl.ds(0, tail)])
            pltpu.sync_copy(rows_hbm.at[pl.ds(off, tail)], rows_v.at[pl.ds(0, tail)])
            adjust(tail)
            for j in range(tail // 16, (tail_groups * _IDXG) // 16):
                idx2_v[j // 8, pl.ds((j % 8) * 16, 16)] = trash
            scatter_groups(tail_groups)

        plsc.subcore_barrier()

        @pl.when(s < r)
        def _():
            a_off = s * (q + 1)
            pltpu.sync_copy(acc.at[pl.ds(a_off, q + 1)],
                            out_hbm.at[pl.ds(nbase + a_off, q + 1)])

        @pl.when(s >= r)
        def _():
            a_off = r * (q + 1) + (s - r) * q
            pltpu.sync_copy(acc.at[pl.ds(a_off, q)],
                            out_hbm.at[pl.ds(nbase + a_off, q)])

    return k


def kernel(node_feat, edge_feat, global_feat, params, edge_idx, node_size):
    n_nodes, _ = node_feat.shape
    n_edges = edge_idx.shape[0]
    ridx = jnp.asarray(edge_idx[:, 0], jnp.int32)
    sidx = jnp.asarray(edge_idx[:, 1], jnp.int32)

    node = _encoder(node_feat, params["node_enc"], 2000)
    edge = _encoder(edge_feat, params["edge_enc"], 2000)

    d = node.shape[1]
    half = n_nodes // 2
    pad = ((half + 8 + 15) // 16) * 16
    zeros_pad = jnp.zeros((pad, d), F32)

    gather = _gather_kernel(n_nodes, n_edges, d)
    scatter = _scatter_kernel(n_nodes, n_edges, d)

    for bp in params["blocks"]:
        r_rows, s_rows = gather(node, ridx, sidx)
        proc, edge = _edge_stage(edge, r_rows, s_rows, bp["edge"], 2000)
        agg = scatter(ridx, proc, zeros_pad)
        node = _node_stage(node, agg, global_feat, bp["node"], 2000)

    pred = _decoder(node, params["node_dec"], 2000)
    return pred + jnp.asarray(node_size - n_nodes, pred.dtype)
